# baseline (speedup 1.0000x reference)
import functools

import jax
import jax.numpy as jnp
from jax.experimental import pallas as pl
from jax.experimental.pallas import tpu as pltpu

_BN = 4096


def _body(x_ref, c_ref, b2_ref, o_ref):
    xb = x_ref[...]
    cs = c_ref[...]
    s = jax.lax.dot_general(
        cs, xb, (((1,), (1,)), ((), ())),
        preferred_element_type=jnp.float32,
    )
    d2 = s + b2_ref[...]
    o_ref[...] = jnp.argmin(d2, axis=0).astype(jnp.int32)


@jax.jit
def kernel(x, centers):
    N, D = x.shape
    K = centers.shape[0]
    cs = centers * (-2.0)
    b2 = jnp.sum(centers * centers, axis=1)[:, None]
    return pl.pallas_call(
        _body,
        grid=(N // _BN,),
        in_specs=[
            pl.BlockSpec((_BN, D), lambda i: (i, 0)),
            pl.BlockSpec((K, D), lambda i: (0, 0)),
            pl.BlockSpec((K, 1), lambda i: (0, 0)),
        ],
        out_specs=pl.BlockSpec((_BN,), lambda i: (i,)),
        out_shape=jax.ShapeDtypeStruct((N,), jnp.int32),
    )(x, cs, b2)


# speedup vs baseline: 1.0322x; 1.0322x over previous
import functools

import jax
import jax.numpy as jnp
from jax.experimental import pallas as pl
from jax.experimental.pallas import tpu as pltpu

_BN = 4096


def _body(x_ref, c_ref, o_ref):
    xb = x_ref[...]
    c = c_ref[...]
    cs = c * (-2.0)
    b2 = jnp.sum(c * c, axis=1, keepdims=True)
    s = jax.lax.dot_general(
        cs, xb, (((1,), (1,)), ((), ())),
        preferred_element_type=jnp.float32,
    )
    d2 = s + b2
    o_ref[...] = jnp.argmin(d2, axis=0).astype(jnp.int32)


@jax.jit
def kernel(x, centers):
    N, D = x.shape
    K = centers.shape[0]
    return pl.pallas_call(
        _body,
        grid=(N // _BN,),
        in_specs=[
            pl.BlockSpec((_BN, D), lambda i: (i, 0)),
            pl.BlockSpec((K, D), lambda i: (0, 0)),
        ],
        out_specs=pl.BlockSpec((_BN,), lambda i: (i,)),
        out_shape=jax.ShapeDtypeStruct((N,), jnp.int32),
    )(x, centers)


# speedup vs baseline: 1.0460x; 1.0134x over previous
import functools

import jax
import jax.numpy as jnp
from jax.experimental import pallas as pl
from jax.experimental.pallas import tpu as pltpu

_BN = 8192


def _body(x_ref, c_ref, o_ref):
    xb = x_ref[...]
    c = c_ref[...]
    cs = c * (-2.0)
    b2 = jnp.sum(c * c, axis=1, keepdims=True)
    s = jax.lax.dot_general(
        cs, xb, (((1,), (1,)), ((), ())),
        preferred_element_type=jnp.float32,
    )
    d2 = s + b2
    o_ref[...] = jnp.argmin(d2, axis=0).astype(jnp.int32)


@jax.jit
def kernel(x, centers):
    N, D = x.shape
    K = centers.shape[0]
    return pl.pallas_call(
        _body,
        grid=(N // _BN,),
        in_specs=[
            pl.BlockSpec((_BN, D), lambda i: (i, 0)),
            pl.BlockSpec((K, D), lambda i: (0, 0)),
        ],
        out_specs=pl.BlockSpec((_BN,), lambda i: (i,)),
        out_shape=jax.ShapeDtypeStruct((N,), jnp.int32),
    )(x, centers)


# speedup vs baseline: 1.0476x; 1.0016x over previous
import jax
import jax.numpy as jnp
from jax.experimental import pallas as pl

_BN = 8192


def _body(x_ref, c_ref, o_ref):
    xb = x_ref[...]
    c = c_ref[...]
    cs = c * (-2.0)
    b2 = jnp.sum(c * c, axis=1, keepdims=True)
    s = jax.lax.dot_general(
        cs, xb, (((1,), (1,)), ((), ())),
        preferred_element_type=jnp.float32,
    )
    d2 = s + b2
    o_ref[...] = jnp.argmin(d2, axis=0).astype(jnp.int32)


@jax.jit
def kernel(x, centers):
    N, D = x.shape
    K = centers.shape[0]
    return pl.pallas_call(
        _body,
        grid=(N // _BN,),
        in_specs=[
            pl.BlockSpec((_BN, D), lambda i: (i, 0)),
            pl.BlockSpec((K, D), lambda i: (0, 0)),
        ],
        out_specs=pl.BlockSpec((_BN,), lambda i: (i,)),
        out_shape=jax.ShapeDtypeStruct((N,), jnp.int32),
    )(x, centers)
